# Initial kernel scaffold; baseline (speedup 1.0000x reference)
import functools

import jax
import jax.numpy as jnp
from jax import lax
from jax.experimental import pallas as pl
from jax.experimental.pallas import tpu as pltpu
from jax.experimental.pallas import tpu_sc as plsc

NV = 10000
NE = 5000
NI = 320000
D = 128

NC = 2
NS = 16
NW = NC * NS
L = 16

CHUNK = 128
CPT = 79
NI_PAD = NW * CPT * CHUNK
CPT16 = NI_PAD // (NS * CHUNK)

HIST = 10240
E_PAD = 5120
V_PAD = 10240
SEG = HIST // NS

_mesh = plsc.VectorSubcoreMesh(
    core_axis_name="c", subcore_axis_name="s", num_cores=NC, num_subcores=NS)


def _fill(ref, n, value):
    v = jnp.full((L,), value, jnp.float32)
    for i in range(n // L):
        ref[pl.ds(i * L, L)] = v


def _fill2(ref, rows, value):
    v = jnp.full((L,), value, jnp.float32)
    for r in range(rows):
        for c in range(D // L):
            ref[r, pl.ds(c * L, L)] = v


def _rsqrt(x):
    i = lax.bitcast_convert_type(x, jnp.int32)
    y = lax.bitcast_convert_type(
        jnp.int32(0x5F3759DF) - lax.shift_right_arithmetic(i, 1), jnp.float32)
    for _ in range(3):
        y = y * (1.5 - 0.5 * x * y * y)
    return y


def _mm_body(x_ref, w_ref, b_ref, o_ref):
    o_ref[...] = (
        jnp.dot(x_ref[...], w_ref[...], preferred_element_type=jnp.float32)
        + b_ref[...])


def _tc_matmul(xp, wt, b2):
    blk = 1024
    return pl.pallas_call(
        _mm_body,
        grid=(V_PAD // blk,),
        in_specs=[
            pl.BlockSpec((blk, D), lambda i: (i, 0)),
            pl.BlockSpec((D, D), lambda i: (0, 0)),
            pl.BlockSpec((1, D), lambda i: (0, 0)),
        ],
        out_specs=pl.BlockSpec((blk, D), lambda i: (i, 0)),
        out_shape=jax.ShapeDtypeStruct((V_PAD, D), jnp.float32),
    )(xp, wt, b2)


@functools.partial(
    pl.kernel,
    out_type=(jax.ShapeDtypeStruct((HIST,), jnp.float32),
              jax.ShapeDtypeStruct((HIST,), jnp.float32)),
    mesh=_mesh,
    scratch_types=[
        pltpu.VMEM((CHUNK,), jnp.int32),
        pltpu.VMEM((CHUNK,), jnp.float32),
        pltpu.VMEM((SEG,), jnp.float32),
        pltpu.VMEM_SHARED((HIST,), jnp.float32),
    ],
)
def _k_hist(src, dst, deg_out, cnt_out, idx, ones, zbuf, hist):
    cid = lax.axis_index("c")
    sid = lax.axis_index("s")
    _fill(ones, CHUNK, 1.0)
    _fill(zbuf, SEG, 0.0)
    pltpu.sync_copy(zbuf, hist.at[pl.ds(sid * SEG, SEG)])
    plsc.subcore_barrier()

    def run(idx_hbm):
        @pl.loop(0, CPT16)
        def _(j):
            base = (sid * CPT16 + j) * CHUNK
            pltpu.sync_copy(idx_hbm.at[pl.ds(base, CHUNK)], idx)
            pltpu.sync_copy(ones, hist.at[idx], add=True)

    @pl.when(cid == 0)
    def _():
        run(src)

    @pl.when(cid == 1)
    def _():
        run(dst)

    plsc.subcore_barrier()

    @pl.when(cid == 0)
    def _():
        pltpu.sync_copy(hist.at[pl.ds(sid * SEG, SEG)],
                        deg_out.at[pl.ds(sid * SEG, SEG)])

    @pl.when(cid == 1)
    def _():
        pltpu.sync_copy(hist.at[pl.ds(sid * SEG, SEG)],
                        cnt_out.at[pl.ds(sid * SEG, SEG)])


@functools.partial(
    pl.kernel,
    out_type=(jax.ShapeDtypeStruct((E_PAD,), jnp.float32),
              jax.ShapeDtypeStruct((HIST,), jnp.float32)),
    mesh=_mesh,
    scratch_types=[
        pltpu.VMEM((CHUNK,), jnp.int32),
        pltpu.VMEM((CHUNK,), jnp.int32),
        pltpu.VMEM((CHUNK,), jnp.float32),
        pltpu.VMEM((SEG,), jnp.float32),
        pltpu.VMEM((SEG,), jnp.float32),
        pltpu.VMEM((SEG,), jnp.float32),
        pltpu.VMEM_SHARED((HIST,), jnp.float32),
        pltpu.SemaphoreType.DMA,
    ],
)
def _k_scales(src, dst, deg_hbm, cnt_hbm, scale_out, dvinv_out,
              sidx, didx, vals, abuf, bbuf, obuf, desum, sem):
    cid = lax.axis_index("c")
    sid = lax.axis_index("s")

    @pl.when(cid == 0)
    def _():
        _fill(obuf, SEG, 0.0)
        pltpu.sync_copy(obuf, desum.at[pl.ds(sid * SEG, SEG)])
        plsc.subcore_barrier()

        @pl.loop(0, CPT16)
        def _(j):
            base = (sid * CPT16 + j) * CHUNK
            pltpu.sync_copy(src.at[pl.ds(base, CHUNK)], sidx)
            pltpu.sync_copy(dst.at[pl.ds(base, CHUNK)], didx)
            pltpu.async_copy(deg_hbm.at[sidx], vals, sem).wait()
            pltpu.sync_copy(vals, desum.at[didx], add=True)

        plsc.subcore_barrier()

        @pl.when(sid < E_PAD // SEG)
        def _():
            pltpu.sync_copy(desum.at[pl.ds(sid * SEG, SEG)], abuf)
            pltpu.sync_copy(cnt_hbm.at[pl.ds(sid * SEG, SEG)], bbuf)
            for k in range(SEG // L):
                s = pl.ds(k * L, L)
                de = abuf[s] / (bbuf[s] + 1.0)
                r = _rsqrt(jnp.maximum(de, 1e-30))
                sc = jnp.where(de > 0, r, 1.0) / jnp.maximum(bbuf[s], 1.0)
                obuf[s] = sc
            pltpu.sync_copy(obuf, scale_out.at[pl.ds(sid * SEG, SEG)])

    @pl.when(cid == 1)
    def _():
        pltpu.sync_copy(deg_hbm.at[pl.ds(sid * SEG, SEG)], abuf)
        for k in range(SEG // L):
            s = pl.ds(k * L, L)
            dg = abuf[s]
            r = _rsqrt(jnp.maximum(dg, 1e-30))
            obuf[s] = jnp.where(dg > 0, r, 0.0)
        pltpu.sync_copy(obuf, dvinv_out.at[pl.ds(sid * SEG, SEG)])


def _make_seg_sum(n_rows):
    rpt = n_rows // NS

    @functools.partial(
        pl.kernel,
        out_type=jax.ShapeDtypeStruct((NC, n_rows, D), jnp.float32),
        mesh=_mesh,
        scratch_types=[
            pltpu.VMEM((CHUNK,), jnp.int32),
            pltpu.VMEM((CHUNK,), jnp.int32),
            pltpu.VMEM((CHUNK, D), jnp.float32),
            pltpu.VMEM((L, D), jnp.float32),
            pltpu.VMEM_SHARED((n_rows, D), jnp.float32),
            pltpu.SemaphoreType.DMA,
        ],
    )
    def seg_sum(gather_idx, scatter_idx, table, out, gidx, scidx, rows, zbuf,
                acc, sem):
        cid = lax.axis_index("c")
        sid = lax.axis_index("s")
        wid = cid * NS + sid
        _fill2(zbuf, L, 0.0)

        @pl.loop(0, rpt // L)
        def _(r):
            pltpu.sync_copy(zbuf, acc.at[pl.ds(sid * rpt + r * L, L)])

        plsc.subcore_barrier()

        @pl.loop(0, CPT)
        def _(j):
            base = (wid * CPT + j) * CHUNK
            pltpu.sync_copy(gather_idx.at[pl.ds(base, CHUNK)], gidx)
            pltpu.sync_copy(scatter_idx.at[pl.ds(base, CHUNK)], scidx)
            pltpu.async_copy(table.at[gidx], rows, sem).wait()
            pltpu.sync_copy(rows, acc.at[scidx], add=True)

        plsc.subcore_barrier()
        pltpu.sync_copy(acc.at[pl.ds(sid * rpt, rpt)],
                        out.at[cid, pl.ds(sid * rpt, rpt)])

    return seg_sum


_k_v2e = _make_seg_sum(E_PAD)
_k_e2v = _make_seg_sum(V_PAD)


def _make_combine(n_rows, relu):
    rpt = n_rows // NW
    rpp = min(rpt, 160)
    npass = rpt // rpp

    @functools.partial(
        pl.kernel,
        out_type=jax.ShapeDtypeStruct((n_rows, D), jnp.float32),
        mesh=_mesh,
        scratch_types=[
            pltpu.VMEM((rpp, D), jnp.float32),
            pltpu.VMEM((rpp, D), jnp.float32),
            pltpu.VMEM((rpt,), jnp.float32),
        ],
    )
    def combine(parts, svec, out, abuf, bbuf, sbuf):
        cid = lax.axis_index("c")
        sid = lax.axis_index("s")
        wid = cid * NS + sid
        base = wid * rpt
        pltpu.sync_copy(svec.at[pl.ds(base, rpt)], sbuf)

        @pl.loop(0, npass)
        def _(p):
            pb = base + p * rpp
            pltpu.sync_copy(parts.at[0, pl.ds(pb, rpp)], abuf)
            pltpu.sync_copy(parts.at[1, pl.ds(pb, rpp)], bbuf)

            @pl.loop(0, rpp)
            def _(r):
                sc = plsc.load_gather(
                    sbuf, [jnp.full((L,), p * rpp, jnp.int32) + r])
                for c in range(D // L):
                    s = pl.ds(c * L, L)
                    v = (abuf[r, s] + bbuf[r, s]) * sc
                    if relu:
                        v = jnp.maximum(v, 0.0)
                    abuf[r, s] = v

            pltpu.sync_copy(abuf, out.at[pl.ds(pb, rpp)])

    return combine


_k_edge_scale = _make_combine(E_PAD, relu=False)
_k_vert_scale = _make_combine(V_PAD, relu=True)


def kernel(X, W, b, v2e_src, v2e_dst):
    src = jnp.concatenate([
        v2e_src.astype(jnp.int32),
        jnp.full((NI_PAD - NI,), NV, jnp.int32)])
    dst = jnp.concatenate([
        v2e_dst.astype(jnp.int32),
        jnp.full((NI_PAD - NI,), NE, jnp.int32)])
    xp = jnp.zeros((V_PAD, D), jnp.float32).at[:NV].set(X)
    xt = _tc_matmul(xp, W.T, b.reshape(1, D))

    deg_v, cnt_e = _k_hist(src, dst)
    scale_e, dv_inv = _k_scales(src, dst, deg_v, cnt_e)

    y_parts = _k_v2e(src, dst, xt)
    y = _k_edge_scale(y_parts, scale_e)

    xo_parts = _k_e2v(dst, src, y)
    xo = _k_vert_scale(xo_parts, dv_inv)
    return xo[:NV]


# speedup vs baseline: 4.7126x; 4.7126x over previous
import functools

import jax
import jax.numpy as jnp
from jax import lax
from jax.experimental import pallas as pl
from jax.experimental.pallas import tpu as pltpu
from jax.experimental.pallas import tpu_sc as plsc

NV = 10000
NE = 5000
NI = 320000
D = 128

NC = 2
NS = 16
NW = NC * NS
L = 16

CHUNK = 128
CPT = 79
NI_PAD = NW * CPT * CHUNK
CPT16 = NI_PAD // (NS * CHUNK)

HIST = 10240
E_PAD = 5120
V_PAD = 10240
SEG = HIST // NS

_mesh = plsc.VectorSubcoreMesh(
    core_axis_name="c", subcore_axis_name="s", num_cores=NC, num_subcores=NS)


def _fill(ref, n, value):
    v = jnp.full((L,), value, jnp.float32)
    for i in range(n // L):
        ref[pl.ds(i * L, L)] = v


def _fill2(ref, rows, value):
    v = jnp.full((L,), value, jnp.float32)
    for r in range(rows):
        for c in range(D // L):
            ref[r, pl.ds(c * L, L)] = v


def _rsqrt(x):
    i = lax.bitcast_convert_type(x, jnp.int32)
    y = lax.bitcast_convert_type(
        jnp.int32(0x5F3759DF) - lax.shift_right_arithmetic(i, 1), jnp.float32)
    for _ in range(3):
        y = y * (1.5 - 0.5 * x * y * y)
    return y


def _mm_body(x_ref, w_ref, b_ref, o_ref):
    o_ref[...] = (
        jnp.dot(x_ref[...], w_ref[...], preferred_element_type=jnp.float32)
        + b_ref[...])


def _tc_matmul(xp, wt, b2):
    blk = 1024
    return pl.pallas_call(
        _mm_body,
        grid=(V_PAD // blk,),
        in_specs=[
            pl.BlockSpec((blk, D), lambda i: (i, 0)),
            pl.BlockSpec((D, D), lambda i: (0, 0)),
            pl.BlockSpec((1, D), lambda i: (0, 0)),
        ],
        out_specs=pl.BlockSpec((blk, D), lambda i: (i, 0)),
        out_shape=jax.ShapeDtypeStruct((V_PAD, D), jnp.float32),
    )(xp, wt, b2)


@functools.partial(
    pl.kernel,
    out_type=(jax.ShapeDtypeStruct((HIST,), jnp.float32),
              jax.ShapeDtypeStruct((HIST,), jnp.float32)),
    mesh=_mesh,
    scratch_types=[
        pltpu.VMEM((CHUNK,), jnp.int32),
        pltpu.VMEM((CHUNK,), jnp.float32),
        pltpu.VMEM((SEG,), jnp.float32),
        pltpu.VMEM_SHARED((HIST,), jnp.float32),
    ],
)
def _k_hist(src, dst, deg_out, cnt_out, idx, ones, zbuf, hist):
    cid = lax.axis_index("c")
    sid = lax.axis_index("s")
    _fill(ones, CHUNK, 1.0)
    _fill(zbuf, SEG, 0.0)
    pltpu.sync_copy(zbuf, hist.at[pl.ds(sid * SEG, SEG)])
    plsc.subcore_barrier()

    def run(idx_hbm):
        @pl.loop(0, CPT16)
        def _(j):
            base = (sid * CPT16 + j) * CHUNK
            pltpu.sync_copy(idx_hbm.at[pl.ds(base, CHUNK)], idx)
            pltpu.sync_copy(ones, hist.at[idx], add=True)

    @pl.when(cid == 0)
    def _():
        run(src)

    @pl.when(cid == 1)
    def _():
        run(dst)

    plsc.subcore_barrier()

    @pl.when(cid == 0)
    def _():
        pltpu.sync_copy(hist.at[pl.ds(sid * SEG, SEG)],
                        deg_out.at[pl.ds(sid * SEG, SEG)])

    @pl.when(cid == 1)
    def _():
        pltpu.sync_copy(hist.at[pl.ds(sid * SEG, SEG)],
                        cnt_out.at[pl.ds(sid * SEG, SEG)])


@functools.partial(
    pl.kernel,
    out_type=(jax.ShapeDtypeStruct((E_PAD,), jnp.float32),
              jax.ShapeDtypeStruct((HIST,), jnp.float32)),
    mesh=_mesh,
    scratch_types=[
        pltpu.VMEM((CHUNK,), jnp.int32),
        pltpu.VMEM((CHUNK,), jnp.int32),
        pltpu.VMEM((CHUNK,), jnp.float32),
        pltpu.VMEM((SEG,), jnp.float32),
        pltpu.VMEM((SEG,), jnp.float32),
        pltpu.VMEM((SEG,), jnp.float32),
        pltpu.VMEM_SHARED((HIST,), jnp.float32),
        pltpu.SemaphoreType.DMA,
    ],
)
def _k_scales(src, dst, deg_hbm, cnt_hbm, scale_out, dvinv_out,
              sidx, didx, vals, abuf, bbuf, obuf, desum, sem):
    cid = lax.axis_index("c")
    sid = lax.axis_index("s")

    @pl.when(cid == 0)
    def _():
        _fill(obuf, SEG, 0.0)
        pltpu.sync_copy(obuf, desum.at[pl.ds(sid * SEG, SEG)])
        plsc.subcore_barrier()

        @pl.loop(0, CPT16)
        def _(j):
            base = (sid * CPT16 + j) * CHUNK
            pltpu.sync_copy(src.at[pl.ds(base, CHUNK)], sidx)
            pltpu.sync_copy(dst.at[pl.ds(base, CHUNK)], didx)
            pltpu.async_copy(deg_hbm.at[sidx], vals, sem).wait()
            pltpu.sync_copy(vals, desum.at[didx], add=True)

        plsc.subcore_barrier()

        @pl.when(sid < E_PAD // SEG)
        def _():
            pltpu.sync_copy(desum.at[pl.ds(sid * SEG, SEG)], abuf)
            pltpu.sync_copy(cnt_hbm.at[pl.ds(sid * SEG, SEG)], bbuf)
            for k in range(SEG // L):
                s = pl.ds(k * L, L)
                de = abuf[s] / (bbuf[s] + 1.0)
                r = _rsqrt(jnp.maximum(de, 1e-30))
                sc = jnp.where(de > 0, r, 1.0) / jnp.maximum(bbuf[s], 1.0)
                obuf[s] = sc
            pltpu.sync_copy(obuf, scale_out.at[pl.ds(sid * SEG, SEG)])

    @pl.when(cid == 1)
    def _():
        pltpu.sync_copy(deg_hbm.at[pl.ds(sid * SEG, SEG)], abuf)
        for k in range(SEG // L):
            s = pl.ds(k * L, L)
            dg = abuf[s]
            r = _rsqrt(jnp.maximum(dg, 1e-30))
            obuf[s] = jnp.where(dg > 0, r, 0.0)
        pltpu.sync_copy(obuf, dvinv_out.at[pl.ds(sid * SEG, SEG)])


def _make_seg_sum(n_rows):
    rpt = n_rows // NS

    @functools.partial(
        pl.kernel,
        out_type=jax.ShapeDtypeStruct((NC, n_rows, D), jnp.float32),
        mesh=_mesh,
        scratch_types=[
            pltpu.VMEM((CHUNK,), jnp.int32),
            pltpu.VMEM((CHUNK,), jnp.int32),
            pltpu.VMEM((CHUNK, D), jnp.float32),
            pltpu.VMEM((L, D), jnp.float32),
            pltpu.VMEM_SHARED((n_rows, D), jnp.float32),
            pltpu.SemaphoreType.DMA,
        ],
    )
    def seg_sum(gather_idx, scatter_idx, table, out, gidx, scidx, rows, zbuf,
                acc, sem):
        cid = lax.axis_index("c")
        sid = lax.axis_index("s")
        wid = cid * NS + sid
        _fill2(zbuf, L, 0.0)

        @pl.loop(0, rpt // L)
        def _(r):
            pltpu.sync_copy(zbuf, acc.at[pl.ds(sid * rpt + r * L, L)])

        plsc.subcore_barrier()

        @pl.loop(0, CPT)
        def _(j):
            base = (wid * CPT + j) * CHUNK
            pltpu.sync_copy(gather_idx.at[pl.ds(base, CHUNK)], gidx)
            pltpu.sync_copy(scatter_idx.at[pl.ds(base, CHUNK)], scidx)
            pltpu.async_copy(table.at[gidx], rows, sem).wait()
            pltpu.sync_copy(rows, acc.at[scidx], add=True)

        plsc.subcore_barrier()
        pltpu.sync_copy(acc.at[pl.ds(sid * rpt, rpt)],
                        out.at[cid, pl.ds(sid * rpt, rpt)])

    return seg_sum


_k_v2e = _make_seg_sum(E_PAD)
_k_e2v = _make_seg_sum(V_PAD)


def _make_combine(n_rows, relu):
    rpt = n_rows // NW
    rpp = min(rpt, 160)
    npass = rpt // rpp

    @functools.partial(
        pl.kernel,
        out_type=jax.ShapeDtypeStruct((n_rows, D), jnp.float32),
        mesh=_mesh,
        scratch_types=[
            pltpu.VMEM((rpp, D), jnp.float32),
            pltpu.VMEM((rpp, D), jnp.float32),
            pltpu.VMEM((rpt,), jnp.float32),
        ],
    )
    def combine(parts, svec, out, abuf, bbuf, sbuf):
        cid = lax.axis_index("c")
        sid = lax.axis_index("s")
        wid = cid * NS + sid
        base = wid * rpt
        pltpu.sync_copy(svec.at[pl.ds(base, rpt)], sbuf)

        @pl.loop(0, npass)
        def _(p):
            pb = base + p * rpp
            pltpu.sync_copy(parts.at[0, pl.ds(pb, rpp)], abuf)
            pltpu.sync_copy(parts.at[1, pl.ds(pb, rpp)], bbuf)

            @pl.loop(0, rpp // L)
            def _(q):
                vrow = sbuf[pl.ds(p * rpp + q * L, L)]
                for j in range(L):
                    sc = vrow[j]
                    r = q * L + j
                    for c in range(D // L):
                        s = pl.ds(c * L, L)
                        v = (abuf[r, s] + bbuf[r, s]) * sc
                        if relu:
                            v = jnp.maximum(v, 0.0)
                        abuf[r, s] = v

            pltpu.sync_copy(abuf, out.at[pl.ds(pb, rpp)])

    return combine


_k_edge_scale = _make_combine(E_PAD, relu=False)
_k_vert_scale = _make_combine(V_PAD, relu=True)


def kernel(X, W, b, v2e_src, v2e_dst):
    src = jnp.concatenate([
        v2e_src.astype(jnp.int32),
        jnp.full((NI_PAD - NI,), NV, jnp.int32)])
    dst = jnp.concatenate([
        v2e_dst.astype(jnp.int32),
        jnp.full((NI_PAD - NI,), NE, jnp.int32)])
    xp = jnp.zeros((V_PAD, D), jnp.float32).at[:NV].set(X)
    xt = _tc_matmul(xp, W.T, b.reshape(1, D))

    deg_v, cnt_e = _k_hist(src, dst)
    scale_e, dv_inv = _k_scales(src, dst, deg_v, cnt_e)

    y_parts = _k_v2e(src, dst, xt)
    y = _k_edge_scale(y_parts, scale_e)

    xo_parts = _k_e2v(dst, src, y)
    xo = _k_vert_scale(xo_parts, dv_inv)
    return xo[:NV]


# speedup vs baseline: 5.2360x; 1.1111x over previous
import functools

import jax
import jax.numpy as jnp
from jax import lax
from jax.experimental import pallas as pl
from jax.experimental.pallas import tpu as pltpu
from jax.experimental.pallas import tpu_sc as plsc

NV = 10000
NE = 5000
NI = 320000
D = 128

NC = 2
NS = 16
NW = NC * NS
L = 16

CHUNK = 128
CPT = 79
NI_PAD = NW * CPT * CHUNK
CPT16 = NI_PAD // (NS * CHUNK)

HIST = 10240
E_PAD = 5120
V_PAD = 10240
SEG = HIST // NS

_mesh = plsc.VectorSubcoreMesh(
    core_axis_name="c", subcore_axis_name="s", num_cores=NC, num_subcores=NS)


def _fill(ref, n, value):
    v = jnp.full((L,), value, jnp.float32)
    for i in range(n // L):
        ref[pl.ds(i * L, L)] = v


def _fill2(ref, rows, value):
    v = jnp.full((L,), value, jnp.float32)
    for r in range(rows):
        for c in range(D // L):
            ref[r, pl.ds(c * L, L)] = v


def _rsqrt(x):
    i = lax.bitcast_convert_type(x, jnp.int32)
    y = lax.bitcast_convert_type(
        jnp.int32(0x5F3759DF) - lax.shift_right_arithmetic(i, 1), jnp.float32)
    for _ in range(3):
        y = y * (1.5 - 0.5 * x * y * y)
    return y


def _mm_body(x_ref, w_ref, b_ref, o_ref):
    o_ref[...] = (
        jnp.dot(x_ref[...], w_ref[...], preferred_element_type=jnp.float32)
        + b_ref[...])


def _tc_matmul(xp, wt, b2):
    blk = 1024
    return pl.pallas_call(
        _mm_body,
        grid=(V_PAD // blk,),
        in_specs=[
            pl.BlockSpec((blk, D), lambda i: (i, 0)),
            pl.BlockSpec((D, D), lambda i: (0, 0)),
            pl.BlockSpec((1, D), lambda i: (0, 0)),
        ],
        out_specs=pl.BlockSpec((blk, D), lambda i: (i, 0)),
        out_shape=jax.ShapeDtypeStruct((V_PAD, D), jnp.float32),
    )(xp, wt, b2)


@functools.partial(
    pl.kernel,
    out_type=(jax.ShapeDtypeStruct((HIST,), jnp.float32),
              jax.ShapeDtypeStruct((HIST,), jnp.float32)),
    mesh=_mesh,
    scratch_types=[
        pltpu.VMEM((CHUNK,), jnp.int32),
        pltpu.VMEM((CHUNK,), jnp.float32),
        pltpu.VMEM((SEG,), jnp.float32),
        pltpu.VMEM_SHARED((HIST,), jnp.float32),
    ],
)
def _k_hist(src, dst, deg_out, cnt_out, idx, ones, zbuf, hist):
    cid = lax.axis_index("c")
    sid = lax.axis_index("s")
    _fill(ones, CHUNK, 1.0)
    _fill(zbuf, SEG, 0.0)
    pltpu.sync_copy(zbuf, hist.at[pl.ds(sid * SEG, SEG)])
    plsc.subcore_barrier()

    def run(idx_hbm):
        @pl.loop(0, CPT16)
        def _(j):
            base = (sid * CPT16 + j) * CHUNK
            pltpu.sync_copy(idx_hbm.at[pl.ds(base, CHUNK)], idx)
            pltpu.sync_copy(ones, hist.at[idx], add=True)

    @pl.when(cid == 0)
    def _():
        run(src)

    @pl.when(cid == 1)
    def _():
        run(dst)

    plsc.subcore_barrier()

    @pl.when(cid == 0)
    def _():
        pltpu.sync_copy(hist.at[pl.ds(sid * SEG, SEG)],
                        deg_out.at[pl.ds(sid * SEG, SEG)])

    @pl.when(cid == 1)
    def _():
        pltpu.sync_copy(hist.at[pl.ds(sid * SEG, SEG)],
                        cnt_out.at[pl.ds(sid * SEG, SEG)])


@functools.partial(
    pl.kernel,
    out_type=(jax.ShapeDtypeStruct((E_PAD,), jnp.float32),
              jax.ShapeDtypeStruct((HIST,), jnp.float32)),
    mesh=_mesh,
    scratch_types=[
        pltpu.VMEM((CHUNK,), jnp.int32),
        pltpu.VMEM((CHUNK,), jnp.int32),
        pltpu.VMEM((CHUNK,), jnp.float32),
        pltpu.VMEM((SEG,), jnp.float32),
        pltpu.VMEM((SEG,), jnp.float32),
        pltpu.VMEM((SEG,), jnp.float32),
        pltpu.VMEM_SHARED((HIST,), jnp.float32),
        pltpu.SemaphoreType.DMA,
    ],
)
def _k_scales(src, dst, deg_hbm, cnt_hbm, scale_out, dvinv_out,
              sidx, didx, vals, abuf, bbuf, obuf, desum, sem):
    cid = lax.axis_index("c")
    sid = lax.axis_index("s")

    @pl.when(cid == 0)
    def _():
        _fill(obuf, SEG, 0.0)
        pltpu.sync_copy(obuf, desum.at[pl.ds(sid * SEG, SEG)])
        plsc.subcore_barrier()

        @pl.loop(0, CPT16)
        def _(j):
            base = (sid * CPT16 + j) * CHUNK
            pltpu.sync_copy(src.at[pl.ds(base, CHUNK)], sidx)
            pltpu.sync_copy(dst.at[pl.ds(base, CHUNK)], didx)
            pltpu.async_copy(deg_hbm.at[sidx], vals, sem).wait()
            pltpu.sync_copy(vals, desum.at[didx], add=True)

        plsc.subcore_barrier()

        @pl.when(sid < E_PAD // SEG)
        def _():
            pltpu.sync_copy(desum.at[pl.ds(sid * SEG, SEG)], abuf)
            pltpu.sync_copy(cnt_hbm.at[pl.ds(sid * SEG, SEG)], bbuf)
            for k in range(SEG // L):
                s = pl.ds(k * L, L)
                de = abuf[s] / (bbuf[s] + 1.0)
                r = _rsqrt(jnp.maximum(de, 1e-30))
                sc = jnp.where(de > 0, r, 1.0) / jnp.maximum(bbuf[s], 1.0)
                obuf[s] = sc
            pltpu.sync_copy(obuf, scale_out.at[pl.ds(sid * SEG, SEG)])

    @pl.when(cid == 1)
    def _():
        pltpu.sync_copy(deg_hbm.at[pl.ds(sid * SEG, SEG)], abuf)
        for k in range(SEG // L):
            s = pl.ds(k * L, L)
            dg = abuf[s]
            r = _rsqrt(jnp.maximum(dg, 1e-30))
            obuf[s] = jnp.where(dg > 0, r, 0.0)
        pltpu.sync_copy(obuf, dvinv_out.at[pl.ds(sid * SEG, SEG)])


def _make_seg_sum(n_rows):
    rpt = n_rows // NS

    @functools.partial(
        pl.kernel,
        out_type=jax.ShapeDtypeStruct((NC, n_rows, D), jnp.float32),
        mesh=_mesh,
        scratch_types=[
            pltpu.VMEM((CHUNK,), jnp.int32),
            pltpu.VMEM((CHUNK,), jnp.int32),
            pltpu.VMEM((CHUNK,), jnp.int32),
            pltpu.VMEM((CHUNK,), jnp.int32),
            pltpu.VMEM((CHUNK, D), jnp.float32),
            pltpu.VMEM((CHUNK, D), jnp.float32),
            pltpu.VMEM((L, D), jnp.float32),
            pltpu.VMEM_SHARED((n_rows, D), jnp.float32),
        ] + [pltpu.SemaphoreType.DMA] * 8,
    )
    def seg_sum(gather_idx, scatter_idx, table, out, g0, g1, s0, s1,
                r0, r1, zbuf, acc, *sems):
        G = [g0, g1]
        S = [s0, s1]
        R = [r0, r1]
        gi = sems[0:2]
        si = sems[2:4]
        gs = sems[4:6]
        ss = sems[6:8]
        cid = lax.axis_index("c")
        sid = lax.axis_index("s")
        wid = cid * NS + sid
        _fill2(zbuf, L, 0.0)

        @pl.loop(0, rpt // L)
        def _(r):
            pltpu.sync_copy(zbuf, acc.at[pl.ds(sid * rpt + r * L, L)])

        plsc.subcore_barrier()

        @pl.loop(0, CPT // 2)
        def _(h):
            j0 = 2 * h
            di = [None] * 4
            for c in range(2):
                base = (wid * CPT + j0 + c) * CHUNK
                di[c] = pltpu.async_copy(
                    gather_idx.at[pl.ds(base, CHUNK)], G[c], gi[c])
                di[2 + c] = pltpu.async_copy(
                    scatter_idx.at[pl.ds(base, CHUNK)], S[c], si[c])
            for d in di:
                d.wait()
            dg = [pltpu.async_copy(table.at[G[c]], R[c], gs[c])
                  for c in range(2)]
            ds = [None] * 2
            for c in range(2):
                dg[c].wait()
                ds[c] = pltpu.async_copy(R[c], acc.at[S[c]], ss[c], add=True)
            for c in range(2):
                ds[c].wait()

        base = (wid * CPT + CPT - 1) * CHUNK
        pltpu.sync_copy(gather_idx.at[pl.ds(base, CHUNK)], g0)
        pltpu.sync_copy(scatter_idx.at[pl.ds(base, CHUNK)], s0)
        pltpu.async_copy(table.at[g0], r0, gs[0]).wait()
        pltpu.sync_copy(r0, acc.at[s0], add=True)

        plsc.subcore_barrier()
        pltpu.sync_copy(acc.at[pl.ds(sid * rpt, rpt)],
                        out.at[cid, pl.ds(sid * rpt, rpt)])

    return seg_sum


_k_v2e = _make_seg_sum(E_PAD)
_k_e2v = _make_seg_sum(V_PAD)


def _make_combine(n_rows, relu):
    rpt = n_rows // NW
    rpp = min(rpt, 160)
    npass = rpt // rpp

    @functools.partial(
        pl.kernel,
        out_type=jax.ShapeDtypeStruct((n_rows, D), jnp.float32),
        mesh=_mesh,
        scratch_types=[
            pltpu.VMEM((rpp, D), jnp.float32),
            pltpu.VMEM((rpp, D), jnp.float32),
            pltpu.VMEM((rpt,), jnp.float32),
        ],
    )
    def combine(parts, svec, out, abuf, bbuf, sbuf):
        cid = lax.axis_index("c")
        sid = lax.axis_index("s")
        wid = cid * NS + sid
        base = wid * rpt
        pltpu.sync_copy(svec.at[pl.ds(base, rpt)], sbuf)

        @pl.loop(0, npass)
        def _(p):
            pb = base + p * rpp
            pltpu.sync_copy(parts.at[0, pl.ds(pb, rpp)], abuf)
            pltpu.sync_copy(parts.at[1, pl.ds(pb, rpp)], bbuf)

            @pl.loop(0, rpp // L)
            def _(q):
                vrow = sbuf[pl.ds(p * rpp + q * L, L)]
                for j in range(L):
                    sc = vrow[j]
                    r = q * L + j
                    for c in range(D // L):
                        s = pl.ds(c * L, L)
                        v = (abuf[r, s] + bbuf[r, s]) * sc
                        if relu:
                            v = jnp.maximum(v, 0.0)
                        abuf[r, s] = v

            pltpu.sync_copy(abuf, out.at[pl.ds(pb, rpp)])

    return combine


_k_edge_scale = _make_combine(E_PAD, relu=False)
_k_vert_scale = _make_combine(V_PAD, relu=True)


def kernel(X, W, b, v2e_src, v2e_dst):
    src = jnp.concatenate([
        v2e_src.astype(jnp.int32),
        jnp.full((NI_PAD - NI,), NV, jnp.int32)])
    dst = jnp.concatenate([
        v2e_dst.astype(jnp.int32),
        jnp.full((NI_PAD - NI,), NE, jnp.int32)])
    xp = jnp.zeros((V_PAD, D), jnp.float32).at[:NV].set(X)
    xt = _tc_matmul(xp, W.T, b.reshape(1, D))

    deg_v, cnt_e = _k_hist(src, dst)
    scale_e, dv_inv = _k_scales(src, dst, deg_v, cnt_e)

    y_parts = _k_v2e(src, dst, xt)
    y = _k_edge_scale(y_parts, scale_e)

    xo_parts = _k_e2v(dst, src, y)
    xo = _k_vert_scale(xo_parts, dv_inv)
    return xo[:NV]


# speedup vs baseline: 6.3041x; 1.2040x over previous
import functools

import jax
import jax.numpy as jnp
from jax import lax
from jax.experimental import pallas as pl
from jax.experimental.pallas import tpu as pltpu
from jax.experimental.pallas import tpu_sc as plsc

NV = 10000
NE = 5000
NI = 320000
D = 128

NC = 2
NS = 16
NW = NC * NS
L = 16

CHUNK = 128
CPT = 79
NI_PAD = NW * CPT * CHUNK
CPT16 = NI_PAD // (NS * CHUNK)

HIST = 10240
E_PAD = 5120
V_PAD = 10240
SEG = HIST // NS

_mesh = plsc.VectorSubcoreMesh(
    core_axis_name="c", subcore_axis_name="s", num_cores=NC, num_subcores=NS)


def _fill(ref, n, value):
    v = jnp.full((L,), value, jnp.float32)
    for i in range(n // L):
        ref[pl.ds(i * L, L)] = v


def _fill2(ref, rows, value):
    v = jnp.full((L,), value, jnp.float32)
    for r in range(rows):
        for c in range(D // L):
            ref[r, pl.ds(c * L, L)] = v


def _rsqrt(x):
    i = lax.bitcast_convert_type(x, jnp.int32)
    y = lax.bitcast_convert_type(
        jnp.int32(0x5F3759DF) - lax.shift_right_arithmetic(i, 1), jnp.float32)
    for _ in range(3):
        y = y * (1.5 - 0.5 * x * y * y)
    return y


def _mm_body(x_ref, w_ref, b_ref, o_ref):
    o_ref[...] = (
        jnp.dot(x_ref[...], w_ref[...], preferred_element_type=jnp.float32)
        + b_ref[...])


def _tc_matmul(xp, wt, b2):
    blk = 1024
    return pl.pallas_call(
        _mm_body,
        grid=(V_PAD // blk,),
        in_specs=[
            pl.BlockSpec((blk, D), lambda i: (i, 0)),
            pl.BlockSpec((D, D), lambda i: (0, 0)),
            pl.BlockSpec((1, D), lambda i: (0, 0)),
        ],
        out_specs=pl.BlockSpec((blk, D), lambda i: (i, 0)),
        out_shape=jax.ShapeDtypeStruct((V_PAD, D), jnp.float32),
    )(xp, wt, b2)


@functools.partial(
    pl.kernel,
    out_type=(jax.ShapeDtypeStruct((HIST,), jnp.float32),
              jax.ShapeDtypeStruct((HIST,), jnp.float32)),
    mesh=_mesh,
    scratch_types=[
        pltpu.VMEM((CHUNK,), jnp.int32),
        pltpu.VMEM((CHUNK,), jnp.int32),
        pltpu.VMEM((CHUNK,), jnp.float32),
        pltpu.VMEM((SEG,), jnp.float32),
        pltpu.VMEM_SHARED((HIST,), jnp.float32),
    ] + [pltpu.SemaphoreType.DMA] * 4,
)
def _k_hist(src, dst, deg_out, cnt_out, i0, i1, ones, zbuf, hist, *sems):
    I = [i0, i1]
    gi = sems[0:2]
    ss = sems[2:4]
    cid = lax.axis_index("c")
    sid = lax.axis_index("s")
    _fill(ones, CHUNK, 1.0)
    _fill(zbuf, SEG, 0.0)
    pltpu.sync_copy(zbuf, hist.at[pl.ds(sid * SEG, SEG)])
    plsc.subcore_barrier()

    def run(idx_hbm):
        @pl.loop(0, CPT16 // 2)
        def _(h):
            di = [None] * 2
            for c in range(2):
                base = (sid * CPT16 + 2 * h + c) * CHUNK
                di[c] = pltpu.async_copy(
                    idx_hbm.at[pl.ds(base, CHUNK)], I[c], gi[c])
            for d in di:
                d.wait()
            ds = [pltpu.async_copy(ones, hist.at[I[c]], ss[c], add=True)
                  for c in range(2)]
            for d in ds:
                d.wait()

    @pl.when(cid == 0)
    def _():
        run(src)

    @pl.when(cid == 1)
    def _():
        run(dst)

    plsc.subcore_barrier()

    @pl.when(cid == 0)
    def _():
        pltpu.sync_copy(hist.at[pl.ds(sid * SEG, SEG)],
                        deg_out.at[pl.ds(sid * SEG, SEG)])

    @pl.when(cid == 1)
    def _():
        pltpu.sync_copy(hist.at[pl.ds(sid * SEG, SEG)],
                        cnt_out.at[pl.ds(sid * SEG, SEG)])


@functools.partial(
    pl.kernel,
    out_type=(jax.ShapeDtypeStruct((E_PAD,), jnp.float32),
              jax.ShapeDtypeStruct((HIST,), jnp.float32)),
    mesh=_mesh,
    scratch_types=[
        pltpu.VMEM((CHUNK,), jnp.int32),
        pltpu.VMEM((CHUNK,), jnp.int32),
        pltpu.VMEM((CHUNK,), jnp.int32),
        pltpu.VMEM((CHUNK,), jnp.int32),
        pltpu.VMEM((CHUNK,), jnp.float32),
        pltpu.VMEM((CHUNK,), jnp.float32),
        pltpu.VMEM((SEG,), jnp.float32),
        pltpu.VMEM((SEG,), jnp.float32),
        pltpu.VMEM((SEG,), jnp.float32),
        pltpu.VMEM_SHARED((HIST,), jnp.float32),
    ] + [pltpu.SemaphoreType.DMA] * 8,
)
def _k_scales(src, dst, deg_hbm, cnt_hbm, scale_out, dvinv_out,
              sx0, sx1, dx0, dx1, v0, v1, abuf, bbuf, obuf, desum, *sems):
    SX = [sx0, sx1]
    DX = [dx0, dx1]
    V = [v0, v1]
    gi = sems[0:2]
    si = sems[2:4]
    gs = sems[4:6]
    ss = sems[6:8]
    cid = lax.axis_index("c")
    sid = lax.axis_index("s")

    @pl.when(cid == 0)
    def _():
        _fill(obuf, SEG, 0.0)
        pltpu.sync_copy(obuf, desum.at[pl.ds(sid * SEG, SEG)])
        plsc.subcore_barrier()

        @pl.loop(0, CPT16 // 2)
        def _(h):
            j0 = 2 * h
            di = [None] * 4
            for c in range(2):
                base = (sid * CPT16 + j0 + c) * CHUNK
                di[c] = pltpu.async_copy(
                    src.at[pl.ds(base, CHUNK)], SX[c], gi[c])
                di[2 + c] = pltpu.async_copy(
                    dst.at[pl.ds(base, CHUNK)], DX[c], si[c])
            for d in di:
                d.wait()
            dg = [pltpu.async_copy(deg_hbm.at[SX[c]], V[c], gs[c])
                  for c in range(2)]
            ds = [None] * 2
            for c in range(2):
                dg[c].wait()
                ds[c] = pltpu.async_copy(V[c], desum.at[DX[c]], ss[c],
                                         add=True)
            for c in range(2):
                ds[c].wait()

        plsc.subcore_barrier()

        @pl.when(sid < E_PAD // SEG)
        def _():
            pltpu.sync_copy(desum.at[pl.ds(sid * SEG, SEG)], abuf)
            pltpu.sync_copy(cnt_hbm.at[pl.ds(sid * SEG, SEG)], bbuf)
            for k in range(SEG // L):
                s = pl.ds(k * L, L)
                de = abuf[s] / (bbuf[s] + 1.0)
                r = _rsqrt(jnp.maximum(de, 1e-30))
                sc = jnp.where(de > 0, r, 1.0) / jnp.maximum(bbuf[s], 1.0)
                obuf[s] = sc
            pltpu.sync_copy(obuf, scale_out.at[pl.ds(sid * SEG, SEG)])

    @pl.when(cid == 1)
    def _():
        pltpu.sync_copy(deg_hbm.at[pl.ds(sid * SEG, SEG)], abuf)
        for k in range(SEG // L):
            s = pl.ds(k * L, L)
            dg = abuf[s]
            r = _rsqrt(jnp.maximum(dg, 1e-30))
            obuf[s] = jnp.where(dg > 0, r, 0.0)
        pltpu.sync_copy(obuf, dvinv_out.at[pl.ds(sid * SEG, SEG)])


def _make_seg_sum(n_rows):
    rpt = n_rows // NS

    @functools.partial(
        pl.kernel,
        out_type=jax.ShapeDtypeStruct((NC, n_rows, D), jnp.float32),
        mesh=_mesh,
        scratch_types=[
            pltpu.VMEM((CHUNK,), jnp.int32),
            pltpu.VMEM((CHUNK,), jnp.int32),
            pltpu.VMEM((CHUNK,), jnp.int32),
            pltpu.VMEM((CHUNK,), jnp.int32),
            pltpu.VMEM((CHUNK, D), jnp.float32),
            pltpu.VMEM((CHUNK, D), jnp.float32),
            pltpu.VMEM((L, D), jnp.float32),
            pltpu.VMEM_SHARED((n_rows, D), jnp.float32),
        ] + [pltpu.SemaphoreType.DMA] * 8,
    )
    def seg_sum(gather_idx, scatter_idx, table, out, g0, g1, s0, s1,
                r0, r1, zbuf, acc, *sems):
        G = [g0, g1]
        S = [s0, s1]
        R = [r0, r1]
        gi = sems[0:2]
        si = sems[2:4]
        gs = sems[4:6]
        ss = sems[6:8]
        cid = lax.axis_index("c")
        sid = lax.axis_index("s")
        wid = cid * NS + sid
        _fill2(zbuf, L, 0.0)

        @pl.loop(0, rpt // L)
        def _(r):
            pltpu.sync_copy(zbuf, acc.at[pl.ds(sid * rpt + r * L, L)])

        plsc.subcore_barrier()

        @pl.loop(0, CPT // 2)
        def _(h):
            j0 = 2 * h
            di = [None] * 4
            for c in range(2):
                base = (wid * CPT + j0 + c) * CHUNK
                di[c] = pltpu.async_copy(
                    gather_idx.at[pl.ds(base, CHUNK)], G[c], gi[c])
                di[2 + c] = pltpu.async_copy(
                    scatter_idx.at[pl.ds(base, CHUNK)], S[c], si[c])
            for d in di:
                d.wait()
            dg = [pltpu.async_copy(table.at[G[c]], R[c], gs[c])
                  for c in range(2)]
            ds = [None] * 2
            for c in range(2):
                dg[c].wait()
                ds[c] = pltpu.async_copy(R[c], acc.at[S[c]], ss[c], add=True)
            for c in range(2):
                ds[c].wait()

        base = (wid * CPT + CPT - 1) * CHUNK
        pltpu.sync_copy(gather_idx.at[pl.ds(base, CHUNK)], g0)
        pltpu.sync_copy(scatter_idx.at[pl.ds(base, CHUNK)], s0)
        pltpu.async_copy(table.at[g0], r0, gs[0]).wait()
        pltpu.sync_copy(r0, acc.at[s0], add=True)

        plsc.subcore_barrier()
        pltpu.sync_copy(acc.at[pl.ds(sid * rpt, rpt)],
                        out.at[cid, pl.ds(sid * rpt, rpt)])

    return seg_sum


_k_v2e = _make_seg_sum(E_PAD)
_k_e2v = _make_seg_sum(V_PAD)


def _make_combine(n_rows, relu):
    rpt = n_rows // NW
    rpp = min(rpt, 160)
    npass = rpt // rpp

    @functools.partial(
        pl.kernel,
        out_type=jax.ShapeDtypeStruct((n_rows, D), jnp.float32),
        mesh=_mesh,
        scratch_types=[
            pltpu.VMEM((rpp, D), jnp.float32),
            pltpu.VMEM((rpp, D), jnp.float32),
            pltpu.VMEM((rpt,), jnp.float32),
        ],
    )
    def combine(parts, svec, out, abuf, bbuf, sbuf):
        cid = lax.axis_index("c")
        sid = lax.axis_index("s")
        wid = cid * NS + sid
        base = wid * rpt
        pltpu.sync_copy(svec.at[pl.ds(base, rpt)], sbuf)

        @pl.loop(0, npass)
        def _(p):
            pb = base + p * rpp
            pltpu.sync_copy(parts.at[0, pl.ds(pb, rpp)], abuf)
            pltpu.sync_copy(parts.at[1, pl.ds(pb, rpp)], bbuf)

            @pl.loop(0, rpp // L)
            def _(q):
                vrow = sbuf[pl.ds(p * rpp + q * L, L)]
                for j in range(L):
                    sc = vrow[j]
                    r = q * L + j
                    for c in range(D // L):
                        s = pl.ds(c * L, L)
                        v = (abuf[r, s] + bbuf[r, s]) * sc
                        if relu:
                            v = jnp.maximum(v, 0.0)
                        abuf[r, s] = v

            pltpu.sync_copy(abuf, out.at[pl.ds(pb, rpp)])

    return combine


_k_edge_scale = _make_combine(E_PAD, relu=False)
_k_vert_scale = _make_combine(V_PAD, relu=True)


def kernel(X, W, b, v2e_src, v2e_dst):
    src = jnp.concatenate([
        v2e_src.astype(jnp.int32),
        jnp.full((NI_PAD - NI,), NV, jnp.int32)])
    dst = jnp.concatenate([
        v2e_dst.astype(jnp.int32),
        jnp.full((NI_PAD - NI,), NE, jnp.int32)])
    xp = jnp.zeros((V_PAD, D), jnp.float32).at[:NV].set(X)
    xt = _tc_matmul(xp, W.T, b.reshape(1, D))

    deg_v, cnt_e = _k_hist(src, dst)
    scale_e, dv_inv = _k_scales(src, dst, deg_v, cnt_e)

    y_parts = _k_v2e(src, dst, xt)
    y = _k_edge_scale(y_parts, scale_e)

    xo_parts = _k_e2v(dst, src, y)
    xo = _k_vert_scale(xo_parts, dv_inv)
    return xo[:NV]
